# Initial kernel scaffold; baseline (speedup 1.0000x reference)
import jax
import jax.numpy as jnp
from jax.experimental import pallas as pl


def kernel(x, edge_index, batch, W1, b1, W2, b2):
    raise NotImplementedError("write your pallas kernel here")


# speedup vs baseline: 13.2163x; 13.2163x over previous
import functools

import jax
import jax.numpy as jnp
from jax import lax
from jax.experimental import pallas as pl
from jax.experimental.pallas import tpu as pltpu
from jax.experimental.pallas import tpu_sc as plsc

N = 10000
E = 320000
D = 128
H = 128
G = 16

NC = 2
NS = 16
NW = NC * NS

CHUNK = 80
EPW = E // NW
NCHUNKS = EPW // CHUNK
RCHUNKS = N // CHUNK

_SC_MESH = plsc.VectorSubcoreMesh(
    core_axis_name="c", subcore_axis_name="s", num_cores=NC, num_subcores=NS)


def _sc_degree(dst):
    @functools.partial(
        pl.kernel,
        out_type=jax.ShapeDtypeStruct((NC * N,), jnp.float32),
        mesh=_SC_MESH,
        scratch_types=[
            pltpu.VMEM((CHUNK,), jnp.int32),
            pltpu.VMEM((CHUNK,), jnp.float32),
            pltpu.VMEM((CHUNK,), jnp.float32),
            pltpu.VMEM_SHARED((N,), jnp.float32),
        ],
    )
    def deg_kernel(dst_hbm, out_hbm, dstbuf, valbuf, zbuf, acc):
        c = lax.axis_index("c")
        s = lax.axis_index("s")
        wid = c * NS + s

        ones16 = jnp.ones((16,), jnp.float32)
        zero16 = jnp.zeros((16,), jnp.float32)

        def fill(i, _):
            valbuf[pl.ds(i * 16, 16)] = ones16
            zbuf[pl.ds(i * 16, 16)] = zero16
            return 0
        lax.fori_loop(0, CHUNK // 16, fill, 0)

        def acc_zero(j, _):
            k = s * 8 + j

            @pl.when(k < RCHUNKS)
            def _():
                pltpu.sync_copy(zbuf, acc.at[pl.ds(k * CHUNK, CHUNK)])
            return 0
        lax.fori_loop(0, 8, acc_zero, 0)
        plsc.subcore_barrier()
        obase = c * N

        base = wid * EPW

        def step(g, _):
            pltpu.sync_copy(dst_hbm.at[pl.ds(base + g * CHUNK, CHUNK)], dstbuf)
            pltpu.sync_copy(valbuf, acc.at[dstbuf], add=True)
            return 0
        lax.fori_loop(0, NCHUNKS, step, 0)
        plsc.subcore_barrier()

        def dump(j, _):
            k = s * 8 + j

            @pl.when(k < RCHUNKS)
            def _():
                pltpu.sync_copy(acc.at[pl.ds(k * CHUNK, CHUNK)], zbuf)
                pltpu.sync_copy(zbuf, out_hbm.at[pl.ds(obase + k * CHUNK, CHUNK)])
            return 0
        lax.fori_loop(0, 8, dump, 0)

    return deg_kernel(dst)


def _sc_scatter(table, src, dst):
    @functools.partial(
        pl.kernel,
        out_type=jax.ShapeDtypeStruct((NC, N, H), jnp.float32),
        mesh=_SC_MESH,
        scratch_types=[
            pltpu.VMEM((CHUNK,), jnp.int32),
            pltpu.VMEM((CHUNK,), jnp.int32),
            pltpu.VMEM((CHUNK, H), jnp.float32),
            pltpu.VMEM_SHARED((N, H), jnp.float32),
        ],
    )
    def scat_kernel(table_hbm, src_hbm, dst_hbm, out_hbm,
                    srcbuf, dstbuf, rows, acc):
        c = lax.axis_index("c")
        s = lax.axis_index("s")
        wid = c * NS + s

        zero16 = jnp.zeros((16,), jnp.float32)

        def zrow(r, _):
            def zcol(cc, _):
                rows[r, pl.ds(cc * 16, 16)] = zero16
                return 0
            lax.fori_loop(0, H // 16, zcol, 0)
            return 0
        lax.fori_loop(0, CHUNK, zrow, 0)

        def acc_zero(j, _):
            k = s * 8 + j

            @pl.when(k < RCHUNKS)
            def _():
                pltpu.sync_copy(rows, acc.at[pl.ds(k * CHUNK, CHUNK), :])
            return 0
        lax.fori_loop(0, 8, acc_zero, 0)
        plsc.subcore_barrier()

        base = wid * EPW

        def step(g, _):
            off = base + g * CHUNK
            pltpu.sync_copy(src_hbm.at[pl.ds(off, CHUNK)], srcbuf)
            pltpu.sync_copy(dst_hbm.at[pl.ds(off, CHUNK)], dstbuf)
            pltpu.sync_copy(table_hbm.at[srcbuf], rows)
            pltpu.sync_copy(rows, acc.at[dstbuf], add=True)
            return 0
        lax.fori_loop(0, NCHUNKS, step, 0)
        plsc.subcore_barrier()

        def dump(j, _):
            k = s * 8 + j

            @pl.when(k < RCHUNKS)
            def _():
                pltpu.sync_copy(acc.at[pl.ds(k * CHUNK, CHUNK), :], rows)
                pltpu.sync_copy(rows, out_hbm.at[c, pl.ds(k * CHUNK, CHUNK), :])
            return 0
        lax.fori_loop(0, 8, dump, 0)

    return scat_kernel(table, src, dst)


_BLK = 1000
_NBLK = N // _BLK


def _tc_prescale(x, W1, degp):
    def body(x_ref, w_ref, dp_ref, hw_ref, dinv_ref):
        deg = 1.0 + dp_ref[0] + dp_ref[1]
        dinv = lax.rsqrt(deg)
        dinv_ref[...] = dinv
        hw_ref[...] = jnp.dot(x_ref[...], w_ref[...],
                              preferred_element_type=jnp.float32) * dinv

    return pl.pallas_call(
        body,
        grid=(_NBLK,),
        in_specs=[
            pl.BlockSpec((_BLK, D), lambda i: (i, 0)),
            pl.BlockSpec((D, H), lambda i: (0, 0)),
            pl.BlockSpec((NC, _BLK, 1), lambda i: (0, i, 0)),
        ],
        out_specs=[
            pl.BlockSpec((_BLK, H), lambda i: (i, 0)),
            pl.BlockSpec((_BLK, 1), lambda i: (i, 0)),
        ],
        out_shape=[
            jax.ShapeDtypeStruct((N, H), jnp.float32),
            jax.ShapeDtypeStruct((N, 1), jnp.float32),
        ],
    )(x, W1, degp)


def _tc_layer_mid(Sp, hwp, dinv, b, W2):
    def body(s_ref, hw_ref, dinv_ref, b_ref, w_ref, out_ref):
        dinv = dinv_ref[...]
        h = s_ref[0] + s_ref[1] + hw_ref[...]
        h = jnp.maximum(dinv * h + b_ref[...], 0.0)
        out_ref[...] = jnp.dot(h, w_ref[...],
                               preferred_element_type=jnp.float32) * dinv

    return pl.pallas_call(
        body,
        grid=(_NBLK,),
        in_specs=[
            pl.BlockSpec((NC, _BLK, H), lambda i: (0, i, 0)),
            pl.BlockSpec((_BLK, H), lambda i: (i, 0)),
            pl.BlockSpec((_BLK, 1), lambda i: (i, 0)),
            pl.BlockSpec((1, H), lambda i: (0, 0)),
            pl.BlockSpec((H, H), lambda i: (0, 0)),
        ],
        out_specs=pl.BlockSpec((_BLK, H), lambda i: (i, 0)),
        out_shape=jax.ShapeDtypeStruct((N, H), jnp.float32),
    )(Sp, hwp, dinv, b, W2)


def _tc_finish_pool(Sp, hwp, dinv, b, batch2d):
    def body(s_ref, hw_ref, dinv_ref, b_ref, bat_ref, out_ref, cnt_ref):
        i = pl.program_id(0)
        dinv = dinv_ref[...]
        h = s_ref[0] + s_ref[1] + hw_ref[...]
        h = jnp.maximum(dinv * h + b_ref[...], 0.0)

        gids = lax.broadcasted_iota(jnp.int32, (_BLK, G), 1)
        onehot = (bat_ref[...] == gids).astype(jnp.float32)
        part = lax.dot_general(onehot, h, (((0,), (0,)), ((), ())),
                               preferred_element_type=jnp.float32)
        pcnt = lax.dot_general(onehot, jnp.ones((_BLK, 1), jnp.float32),
                               (((0,), (0,)), ((), ())),
                               preferred_element_type=jnp.float32)

        @pl.when(i == 0)
        def _():
            out_ref[...] = jnp.zeros_like(out_ref)
            cnt_ref[...] = jnp.zeros_like(cnt_ref)

        out_ref[...] += part
        cnt_ref[...] += pcnt

        @pl.when(i == _NBLK - 1)
        def _():
            out_ref[...] = out_ref[...] / jnp.maximum(cnt_ref[...], 1.0)

    return pl.pallas_call(
        body,
        grid=(_NBLK,),
        in_specs=[
            pl.BlockSpec((NC, _BLK, H), lambda i: (0, i, 0)),
            pl.BlockSpec((_BLK, H), lambda i: (i, 0)),
            pl.BlockSpec((_BLK, 1), lambda i: (i, 0)),
            pl.BlockSpec((1, H), lambda i: (0, 0)),
            pl.BlockSpec((_BLK, 1), lambda i: (i, 0)),
        ],
        out_specs=pl.BlockSpec((G, H), lambda i: (0, 0)),
        out_shape=jax.ShapeDtypeStruct((G, H), jnp.float32),
        scratch_shapes=[pltpu.VMEM((G, 1), jnp.float32)],
    )(Sp, hwp, dinv, b, batch2d)


def kernel(x, edge_index, batch, W1, b1, W2, b2):
    src = edge_index[0]
    dst = edge_index[1]

    degp = _sc_degree(dst)
    degp3 = degp.reshape(NC, N, 1)

    hw1p, dinv = _tc_prescale(x, W1, degp3)
    S1 = _sc_scatter(hw1p, src, dst)
    hw2p = _tc_layer_mid(S1, hw1p, dinv, b1.reshape(1, H), W2)
    S2 = _sc_scatter(hw2p, src, dst)
    g = _tc_finish_pool(S2, hw2p, dinv, b2.reshape(1, H),
                        batch.reshape(N, 1))
    return g


# speedup vs baseline: 23.8421x; 1.8040x over previous
import functools

import jax
import jax.numpy as jnp
from jax import lax
from jax.experimental import pallas as pl
from jax.experimental.pallas import tpu as pltpu
from jax.experimental.pallas import tpu_sc as plsc

N = 10000
E = 320000
D = 128
H = 128
G = 16

NC = 2
NS = 16
NW = NC * NS

CHUNK = 128
EPW = E // NW
NFULL = EPW // CHUNK
TAIL = EPW - NFULL * CHUNK
NPAIR = NFULL // 2
ZCHUNK = 80
RCHUNKS = N // ZCHUNK

_SC_MESH = plsc.VectorSubcoreMesh(
    core_axis_name="c", subcore_axis_name="s", num_cores=NC, num_subcores=NS)


def _sc_degree(dst):
    @functools.partial(
        pl.kernel,
        out_type=jax.ShapeDtypeStruct((NC * N,), jnp.float32),
        mesh=_SC_MESH,
        scratch_types=[
            pltpu.VMEM((CHUNK,), jnp.int32),
            pltpu.VMEM((CHUNK,), jnp.int32),
            pltpu.VMEM((TAIL,), jnp.int32),
            pltpu.VMEM((CHUNK,), jnp.float32),
            pltpu.VMEM((ZCHUNK,), jnp.float32),
            pltpu.VMEM_SHARED((N,), jnp.float32),
            pltpu.SemaphoreType.DMA,
            pltpu.SemaphoreType.DMA,
        ],
    )
    def deg_kernel(dst_hbm, out_hbm, dstbuf0, dstbuf1, dstT, valbuf, zbuf,
                   acc, sem0, sem1):
        c = lax.axis_index("c")
        s = lax.axis_index("s")
        wid = c * NS + s

        ones16 = jnp.ones((16,), jnp.float32)
        zero16 = jnp.zeros((16,), jnp.float32)

        def fill(i, _):
            valbuf[pl.ds(i * 16, 16)] = ones16
            return 0
        lax.fori_loop(0, CHUNK // 16, fill, 0)

        def zfill(i, _):
            zbuf[pl.ds(i * 16, 16)] = zero16
            return 0
        lax.fori_loop(0, ZCHUNK // 16, zfill, 0)

        def acc_zero(j, _):
            k = s * 8 + j

            @pl.when(k < RCHUNKS)
            def _():
                pltpu.sync_copy(zbuf, acc.at[pl.ds(k * ZCHUNK, ZCHUNK)])
            return 0
        lax.fori_loop(0, 8, acc_zero, 0)
        plsc.subcore_barrier()

        base = wid * EPW

        def pair(j, _):
            o = base + 2 * j * CHUNK
            pltpu.sync_copy(dst_hbm.at[pl.ds(o, CHUNK)], dstbuf0)
            s0 = pltpu.async_copy(valbuf, acc.at[dstbuf0], sem0, add=True)
            pltpu.sync_copy(dst_hbm.at[pl.ds(o + CHUNK, CHUNK)], dstbuf1)
            s1 = pltpu.async_copy(valbuf, acc.at[dstbuf1], sem1, add=True)
            s0.wait()
            s1.wait()
            return 0
        lax.fori_loop(0, NPAIR, pair, 0)

        pltpu.sync_copy(dst_hbm.at[pl.ds(base + NFULL * CHUNK, TAIL)], dstT)
        pltpu.sync_copy(valbuf.at[pl.ds(0, TAIL)], acc.at[dstT], add=True)
        plsc.subcore_barrier()

        obase = c * N

        def dump(j, _):
            k = s * 8 + j

            @pl.when(k < RCHUNKS)
            def _():
                pltpu.sync_copy(acc.at[pl.ds(k * ZCHUNK, ZCHUNK)], zbuf)
                pltpu.sync_copy(zbuf, out_hbm.at[pl.ds(obase + k * ZCHUNK, ZCHUNK)])
            return 0
        lax.fori_loop(0, 8, dump, 0)

    return deg_kernel(dst)


def _sc_scatter(table, src, dst):
    @functools.partial(
        pl.kernel,
        out_type=jax.ShapeDtypeStruct((NC, N, H), jnp.float32),
        mesh=_SC_MESH,
        scratch_types=[
            pltpu.VMEM((CHUNK,), jnp.int32),
            pltpu.VMEM((CHUNK,), jnp.int32),
            pltpu.VMEM((CHUNK,), jnp.int32),
            pltpu.VMEM((CHUNK,), jnp.int32),
            pltpu.VMEM((TAIL,), jnp.int32),
            pltpu.VMEM((TAIL,), jnp.int32),
            pltpu.VMEM((CHUNK, H), jnp.float32),
            pltpu.VMEM((CHUNK, H), jnp.float32),
            pltpu.VMEM((TAIL, H), jnp.float32),
            pltpu.VMEM_SHARED((N, H), jnp.float32),
            pltpu.SemaphoreType.DMA,
            pltpu.SemaphoreType.DMA,
            pltpu.SemaphoreType.DMA,
            pltpu.SemaphoreType.DMA,
        ],
    )
    def scat_kernel(table_hbm, src_hbm, dst_hbm, out_hbm,
                    srcbuf0, srcbuf1, dstbuf0, dstbuf1, srcT, dstT,
                    rows0, rows1, rowsT, acc, gsem0, gsem1, ssem0, ssem1):
        c = lax.axis_index("c")
        s = lax.axis_index("s")
        wid = c * NS + s

        zero16 = jnp.zeros((16,), jnp.float32)

        def zrow(r, _):
            def zcol(cc, _):
                rows0[r, pl.ds(cc * 16, 16)] = zero16
                return 0
            lax.fori_loop(0, H // 16, zcol, 0)
            return 0
        lax.fori_loop(0, ZCHUNK, zrow, 0)

        def acc_zero(j, _):
            k = s * 8 + j

            @pl.when(k < RCHUNKS)
            def _():
                pltpu.sync_copy(rows0.at[pl.ds(0, ZCHUNK), :],
                                acc.at[pl.ds(k * ZCHUNK, ZCHUNK), :])
            return 0
        lax.fori_loop(0, 8, acc_zero, 0)
        plsc.subcore_barrier()

        base = wid * EPW

        def load_idx(chunk_i, sbuf, dbuf):
            off = base + chunk_i * CHUNK
            pltpu.sync_copy(src_hbm.at[pl.ds(off, CHUNK)], sbuf)
            pltpu.sync_copy(dst_hbm.at[pl.ds(off, CHUNK)], dbuf)

        load_idx(0, srcbuf0, dstbuf0)
        pltpu.async_copy(table_hbm.at[srcbuf0], rows0, gsem0)
        load_idx(1, srcbuf1, dstbuf1)
        pltpu.async_copy(table_hbm.at[srcbuf1], rows1, gsem1)

        def pair(j, _):
            pltpu.make_async_copy(table_hbm.at[srcbuf0], rows0, gsem0).wait()
            s0 = pltpu.async_copy(rows0, acc.at[dstbuf0], ssem0, add=True)
            pltpu.make_async_copy(table_hbm.at[srcbuf1], rows1, gsem1).wait()
            s1 = pltpu.async_copy(rows1, acc.at[dstbuf1], ssem1, add=True)

            @pl.when(j < NPAIR - 1)
            def _():
                s0.wait()
                load_idx(2 * j + 2, srcbuf0, dstbuf0)
                pltpu.async_copy(table_hbm.at[srcbuf0], rows0, gsem0)
                s1.wait()
                load_idx(2 * j + 3, srcbuf1, dstbuf1)
                pltpu.async_copy(table_hbm.at[srcbuf1], rows1, gsem1)

            @pl.when(j == NPAIR - 1)
            def _():
                s0.wait()
                s1.wait()
            return 0
        lax.fori_loop(0, NPAIR, pair, 0)

        toff = base + NFULL * CHUNK
        pltpu.sync_copy(src_hbm.at[pl.ds(toff, TAIL)], srcT)
        pltpu.sync_copy(dst_hbm.at[pl.ds(toff, TAIL)], dstT)
        pltpu.sync_copy(table_hbm.at[srcT], rowsT)
        pltpu.sync_copy(rowsT, acc.at[dstT], add=True)
        plsc.subcore_barrier()

        def dump(j, _):
            k = s * 8 + j

            @pl.when(k < RCHUNKS)
            def _():
                pltpu.sync_copy(acc.at[pl.ds(k * ZCHUNK, ZCHUNK), :],
                                rows0.at[pl.ds(0, ZCHUNK), :])
                pltpu.sync_copy(rows0.at[pl.ds(0, ZCHUNK), :],
                                out_hbm.at[c, pl.ds(k * ZCHUNK, ZCHUNK), :])
            return 0
        lax.fori_loop(0, 8, dump, 0)

    return scat_kernel(table, src, dst)


_BLK = 1000
_NBLK = N // _BLK


def _tc_prescale(x, W1, degp):
    def body(x_ref, w_ref, dp_ref, hw_ref, dinv_ref):
        deg = 1.0 + dp_ref[0] + dp_ref[1]
        dinv = lax.rsqrt(deg)
        dinv_ref[...] = dinv
        hw_ref[...] = jnp.dot(x_ref[...], w_ref[...],
                              preferred_element_type=jnp.float32) * dinv

    return pl.pallas_call(
        body,
        grid=(_NBLK,),
        in_specs=[
            pl.BlockSpec((_BLK, D), lambda i: (i, 0)),
            pl.BlockSpec((D, H), lambda i: (0, 0)),
            pl.BlockSpec((NC, _BLK, 1), lambda i: (0, i, 0)),
        ],
        out_specs=[
            pl.BlockSpec((_BLK, H), lambda i: (i, 0)),
            pl.BlockSpec((_BLK, 1), lambda i: (i, 0)),
        ],
        out_shape=[
            jax.ShapeDtypeStruct((N, H), jnp.float32),
            jax.ShapeDtypeStruct((N, 1), jnp.float32),
        ],
    )(x, W1, degp)


def _tc_layer_mid(Sp, hwp, dinv, b, W2):
    def body(s_ref, hw_ref, dinv_ref, b_ref, w_ref, out_ref):
        dinv = dinv_ref[...]
        h = s_ref[0] + s_ref[1] + hw_ref[...]
        h = jnp.maximum(dinv * h + b_ref[...], 0.0)
        out_ref[...] = jnp.dot(h, w_ref[...],
                               preferred_element_type=jnp.float32) * dinv

    return pl.pallas_call(
        body,
        grid=(_NBLK,),
        in_specs=[
            pl.BlockSpec((NC, _BLK, H), lambda i: (0, i, 0)),
            pl.BlockSpec((_BLK, H), lambda i: (i, 0)),
            pl.BlockSpec((_BLK, 1), lambda i: (i, 0)),
            pl.BlockSpec((1, H), lambda i: (0, 0)),
            pl.BlockSpec((H, H), lambda i: (0, 0)),
        ],
        out_specs=pl.BlockSpec((_BLK, H), lambda i: (i, 0)),
        out_shape=jax.ShapeDtypeStruct((N, H), jnp.float32),
    )(Sp, hwp, dinv, b, W2)


def _tc_finish_pool(Sp, hwp, dinv, b, batch2d):
    def body(s_ref, hw_ref, dinv_ref, b_ref, bat_ref, out_ref, cnt_ref):
        i = pl.program_id(0)
        dinv = dinv_ref[...]
        h = s_ref[0] + s_ref[1] + hw_ref[...]
        h = jnp.maximum(dinv * h + b_ref[...], 0.0)

        gids = lax.broadcasted_iota(jnp.int32, (_BLK, G), 1)
        onehot = (bat_ref[...] == gids).astype(jnp.float32)
        part = lax.dot_general(onehot, h, (((0,), (0,)), ((), ())),
                               preferred_element_type=jnp.float32)
        pcnt = lax.dot_general(onehot, jnp.ones((_BLK, 1), jnp.float32),
                               (((0,), (0,)), ((), ())),
                               preferred_element_type=jnp.float32)

        @pl.when(i == 0)
        def _():
            out_ref[...] = jnp.zeros_like(out_ref)
            cnt_ref[...] = jnp.zeros_like(cnt_ref)

        out_ref[...] += part
        cnt_ref[...] += pcnt

        @pl.when(i == _NBLK - 1)
        def _():
            out_ref[...] = out_ref[...] / jnp.maximum(cnt_ref[...], 1.0)

    return pl.pallas_call(
        body,
        grid=(_NBLK,),
        in_specs=[
            pl.BlockSpec((NC, _BLK, H), lambda i: (0, i, 0)),
            pl.BlockSpec((_BLK, H), lambda i: (i, 0)),
            pl.BlockSpec((_BLK, 1), lambda i: (i, 0)),
            pl.BlockSpec((1, H), lambda i: (0, 0)),
            pl.BlockSpec((_BLK, 1), lambda i: (i, 0)),
        ],
        out_specs=pl.BlockSpec((G, H), lambda i: (0, 0)),
        out_shape=jax.ShapeDtypeStruct((G, H), jnp.float32),
        scratch_shapes=[pltpu.VMEM((G, 1), jnp.float32)],
    )(Sp, hwp, dinv, b, batch2d)


def kernel(x, edge_index, batch, W1, b1, W2, b2):
    src = edge_index[0]
    dst = edge_index[1]

    degp = _sc_degree(dst)
    degp3 = degp.reshape(NC, N, 1)

    hw1p, dinv = _tc_prescale(x, W1, degp3)
    S1 = _sc_scatter(hw1p, src, dst)
    hw2p = _tc_layer_mid(S1, hw1p, dinv, b1.reshape(1, H), W2)
    S2 = _sc_scatter(hw2p, src, dst)
    g = _tc_finish_pool(S2, hw2p, dinv, b2.reshape(1, H),
                        batch.reshape(N, 1))
    return g


# speedup vs baseline: 25.3931x; 1.0651x over previous
import functools

import jax
import jax.numpy as jnp
from jax import lax
from jax.experimental import pallas as pl
from jax.experimental.pallas import tpu as pltpu
from jax.experimental.pallas import tpu_sc as plsc

N = 10000
E = 320000
D = 128
H = 128
G = 16

NC = 2
NS = 16
NW = NC * NS

CHUNK = 80
EPW = E // NW
NFULL = EPW // CHUNK
NPAIR = NFULL // 2
NLEFT = NFULL - 2 * NPAIR
ZCHUNK = 80
RCHUNKS = N // ZCHUNK

_SC_MESH = plsc.VectorSubcoreMesh(
    core_axis_name="c", subcore_axis_name="s", num_cores=NC, num_subcores=NS)


def _sc_degree(dstm):
    @functools.partial(
        pl.kernel,
        out_type=jax.ShapeDtypeStruct((NC * N,), jnp.float32),
        mesh=_SC_MESH,
        scratch_types=[
            pltpu.VMEM((NFULL, CHUNK), jnp.int32),
            pltpu.VMEM((CHUNK,), jnp.float32),
            pltpu.VMEM((ZCHUNK,), jnp.float32),
            pltpu.VMEM_SHARED((N,), jnp.float32),
            pltpu.SemaphoreType.DMA,
        ],
    )
    def deg_kernel(dstm_hbm, out_hbm, dstidx, valbuf, zbuf, acc, sem):
        c = lax.axis_index("c")
        s = lax.axis_index("s")
        wid = c * NS + s

        ones16 = jnp.ones((16,), jnp.float32)
        zero16 = jnp.zeros((16,), jnp.float32)

        def fill(i, _):
            valbuf[pl.ds(i * 16, 16)] = ones16
            return 0
        lax.fori_loop(0, CHUNK // 16, fill, 0)

        def zfill(i, _):
            zbuf[pl.ds(i * 16, 16)] = zero16
            return 0
        lax.fori_loop(0, ZCHUNK // 16, zfill, 0)

        pltpu.sync_copy(dstm_hbm.at[wid], dstidx)

        def acc_zero(j, _):
            k = s * 8 + j

            @pl.when(k < RCHUNKS)
            def _():
                pltpu.sync_copy(zbuf, acc.at[pl.ds(k * ZCHUNK, ZCHUNK)])
            return 0
        lax.fori_loop(0, 8, acc_zero, 0)
        plsc.subcore_barrier()

        def fire(g, _):
            pltpu.async_copy(valbuf, acc.at[dstidx.at[g]], sem, add=True)
            return 0
        lax.fori_loop(0, NFULL, fire, 0)

        def drain(g, _):
            pltpu.make_async_copy(valbuf, acc.at[dstidx.at[0]], sem).wait()
            return 0
        lax.fori_loop(0, NFULL, drain, 0)

        plsc.subcore_barrier()

        obase = c * N

        def dump(j, _):
            k = s * 8 + j

            @pl.when(k < RCHUNKS)
            def _():
                pltpu.sync_copy(acc.at[pl.ds(k * ZCHUNK, ZCHUNK)], zbuf)
                pltpu.sync_copy(zbuf, out_hbm.at[pl.ds(obase + k * ZCHUNK, ZCHUNK)])
            return 0
        lax.fori_loop(0, 8, dump, 0)

    return deg_kernel(dstm)


NBUF = 2


def _sc_scatter(table, srcm, dstm):
    @functools.partial(
        pl.kernel,
        out_type=jax.ShapeDtypeStruct((NC, N, H), jnp.float32),
        mesh=_SC_MESH,
        scratch_types=[
            pltpu.VMEM((EPW,), jnp.int32),
            pltpu.VMEM((NFULL, CHUNK), jnp.int32),
            [pltpu.VMEM((CHUNK, H), jnp.float32)] * NBUF,
            pltpu.VMEM_SHARED((N, H), jnp.float32),
            [pltpu.SemaphoreType.DMA] * NBUF,
            [pltpu.SemaphoreType.DMA] * NBUF,
        ],
    )
    def scat_kernel(table_hbm, srcm_hbm, dstm_hbm,
                    out_hbm, srcidx, dstidx, rows, acc, gsems, ssems):
        c = lax.axis_index("c")
        s = lax.axis_index("s")
        wid = c * NS + s

        zero16 = jnp.zeros((16,), jnp.float32)

        pltpu.sync_copy(srcm_hbm.at[pl.ds(wid * EPW, EPW)], srcidx)
        pltpu.sync_copy(dstm_hbm.at[wid], dstidx)

        def zrow(r, _):
            def zcol(cc, _):
                rows[0][r, pl.ds(cc * 16, 16)] = zero16
                return 0
            lax.fori_loop(0, H // 16, zcol, 0)
            return 0
        lax.fori_loop(0, ZCHUNK, zrow, 0)

        def acc_zero(j, _):
            k = s * 8 + j

            @pl.when(k < RCHUNKS)
            def _():
                pltpu.sync_copy(rows[0].at[pl.ds(0, ZCHUNK), :],
                                acc.at[pl.ds(k * ZCHUNK, ZCHUNK), :])
            return 0
        lax.fori_loop(0, 8, acc_zero, 0)
        plsc.subcore_barrier()

        def gather(g, b):
            pltpu.async_copy(table_hbm.at[srcidx.at[pl.ds(g * CHUNK, CHUNK)]],
                             rows[b], gsems[b])

        def gather_wait(b):
            pltpu.make_async_copy(
                table_hbm.at[srcidx.at[pl.ds(0, CHUNK)]], rows[b],
                gsems[b]).wait()

        def scatter(g, b):
            pltpu.async_copy(rows[b], acc.at[dstidx.at[g]], ssems[b],
                             add=True)

        def scatter_wait(b):
            pltpu.make_async_copy(rows[b], acc.at[dstidx.at[0]],
                                  ssems[b]).wait()

        for b in range(NBUF):
            gather(b, b)

        def pair(j, _):
            for b in range(NBUF):
                gather_wait(b)
                scatter(NBUF * j + b, b)

            @pl.when(j < NPAIR - 1)
            def _():
                for b in range(NBUF):
                    scatter_wait(b)
                    gather(NBUF * j + NBUF + b, b)

            @pl.when(j == NPAIR - 1)
            def _():
                for b in range(NBUF):
                    scatter_wait(b)
            return 0
        lax.fori_loop(0, NPAIR, pair, 0)

        for i in range(NLEFT):
            g = 2 * NPAIR + i
            gather(g, i)
            gather_wait(i)
            scatter(g, i)
            scatter_wait(i)
        plsc.subcore_barrier()

        def dump(j, _):
            k = s * 8 + j

            @pl.when(k < RCHUNKS)
            def _():
                pltpu.sync_copy(acc.at[pl.ds(k * ZCHUNK, ZCHUNK), :],
                                rows[0].at[pl.ds(0, ZCHUNK), :])
                pltpu.sync_copy(rows[0].at[pl.ds(0, ZCHUNK), :],
                                out_hbm.at[c, pl.ds(k * ZCHUNK, ZCHUNK), :])
            return 0
        lax.fori_loop(0, 8, dump, 0)

    return scat_kernel(table, srcm, dstm)


_BLK = 1000
_NBLK = N // _BLK


def _tc_prescale(x, W1, degp):
    def body(x_ref, w_ref, dp_ref, hw_ref, dinv_ref):
        deg = 1.0 + dp_ref[0] + dp_ref[1]
        dinv = lax.rsqrt(deg)
        dinv_ref[...] = dinv
        hw_ref[...] = jnp.dot(x_ref[...], w_ref[...],
                              preferred_element_type=jnp.float32) * dinv

    return pl.pallas_call(
        body,
        grid=(_NBLK,),
        in_specs=[
            pl.BlockSpec((_BLK, D), lambda i: (i, 0)),
            pl.BlockSpec((D, H), lambda i: (0, 0)),
            pl.BlockSpec((NC, _BLK, 1), lambda i: (0, i, 0)),
        ],
        out_specs=[
            pl.BlockSpec((_BLK, H), lambda i: (i, 0)),
            pl.BlockSpec((_BLK, 1), lambda i: (i, 0)),
        ],
        out_shape=[
            jax.ShapeDtypeStruct((N, H), jnp.float32),
            jax.ShapeDtypeStruct((N, 1), jnp.float32),
        ],
    )(x, W1, degp)


def _tc_layer_mid(Sp, hwp, dinv, b, W2):
    def body(s_ref, hw_ref, dinv_ref, b_ref, w_ref, out_ref):
        dinv = dinv_ref[...]
        h = s_ref[0] + s_ref[1] + hw_ref[...]
        h = jnp.maximum(dinv * h + b_ref[...], 0.0)
        out_ref[...] = jnp.dot(h, w_ref[...],
                               preferred_element_type=jnp.float32) * dinv

    return pl.pallas_call(
        body,
        grid=(_NBLK,),
        in_specs=[
            pl.BlockSpec((NC, _BLK, H), lambda i: (0, i, 0)),
            pl.BlockSpec((_BLK, H), lambda i: (i, 0)),
            pl.BlockSpec((_BLK, 1), lambda i: (i, 0)),
            pl.BlockSpec((1, H), lambda i: (0, 0)),
            pl.BlockSpec((H, H), lambda i: (0, 0)),
        ],
        out_specs=pl.BlockSpec((_BLK, H), lambda i: (i, 0)),
        out_shape=jax.ShapeDtypeStruct((N, H), jnp.float32),
    )(Sp, hwp, dinv, b, W2)


def _tc_finish_pool(Sp, hwp, dinv, b, batch2d):
    def body(s_ref, hw_ref, dinv_ref, b_ref, bat_ref, out_ref, cnt_ref):
        i = pl.program_id(0)
        dinv = dinv_ref[...]
        h = s_ref[0] + s_ref[1] + hw_ref[...]
        h = jnp.maximum(dinv * h + b_ref[...], 0.0)

        gids = lax.broadcasted_iota(jnp.int32, (_BLK, G), 1)
        onehot = (bat_ref[...] == gids).astype(jnp.float32)
        part = lax.dot_general(onehot, h, (((0,), (0,)), ((), ())),
                               preferred_element_type=jnp.float32)
        pcnt = lax.dot_general(onehot, jnp.ones((_BLK, 1), jnp.float32),
                               (((0,), (0,)), ((), ())),
                               preferred_element_type=jnp.float32)

        @pl.when(i == 0)
        def _():
            out_ref[...] = jnp.zeros_like(out_ref)
            cnt_ref[...] = jnp.zeros_like(cnt_ref)

        out_ref[...] += part
        cnt_ref[...] += pcnt

        @pl.when(i == _NBLK - 1)
        def _():
            out_ref[...] = out_ref[...] / jnp.maximum(cnt_ref[...], 1.0)

    return pl.pallas_call(
        body,
        grid=(_NBLK,),
        in_specs=[
            pl.BlockSpec((NC, _BLK, H), lambda i: (0, i, 0)),
            pl.BlockSpec((_BLK, H), lambda i: (i, 0)),
            pl.BlockSpec((_BLK, 1), lambda i: (i, 0)),
            pl.BlockSpec((1, H), lambda i: (0, 0)),
            pl.BlockSpec((_BLK, 1), lambda i: (i, 0)),
        ],
        out_specs=pl.BlockSpec((G, H), lambda i: (0, 0)),
        out_shape=jax.ShapeDtypeStruct((G, H), jnp.float32),
        scratch_shapes=[pltpu.VMEM((G, 1), jnp.float32)],
    )(Sp, hwp, dinv, b, batch2d)


def kernel(x, edge_index, batch, W1, b1, W2, b2):
    srcf = edge_index[0]
    dstm = edge_index[1].reshape(NW, NFULL, CHUNK)

    degp = _sc_degree(dstm)
    degp3 = degp.reshape(NC, N, 1)

    hw1p, dinv = _tc_prescale(x, W1, degp3)
    S1 = _sc_scatter(hw1p, srcf, dstm)
    hw2p = _tc_layer_mid(S1, hw1p, dinv, b1.reshape(1, H), W2)
    S2 = _sc_scatter(hw2p, srcf, dstm)
    g = _tc_finish_pool(S2, hw2p, dinv, b2.reshape(1, H),
                        batch.reshape(N, 1))
    return g


# speedup vs baseline: 30.0194x; 1.1822x over previous
import functools

import jax
import jax.numpy as jnp
from jax import lax
from jax.experimental import pallas as pl
from jax.experimental.pallas import tpu as pltpu
from jax.experimental.pallas import tpu_sc as plsc

N = 10000
E = 320000
D = 128
H = 128
G = 16

NC = 2
NS = 16
NW = NC * NS

CHUNK = 80
EPW = E // NW
NFULL = EPW // CHUNK
NBUF = 3
PH0 = 63
PH1 = NFULL - PH0
NR0 = PH0 // NBUF
NR1 = PH1 // NBUF
NLEFT = PH1 - NR1 * NBUF
ZCHUNK = 80
RCHUNKS = N // ZCHUNK

_SC_MESH = plsc.VectorSubcoreMesh(
    core_axis_name="c", subcore_axis_name="s", num_cores=NC, num_subcores=NS)


def _sc_degree(dstm):
    @functools.partial(
        pl.kernel,
        out_type=jax.ShapeDtypeStruct((NC * N,), jnp.float32),
        mesh=_SC_MESH,
        scratch_types=[
            pltpu.VMEM((NFULL, CHUNK), jnp.int32),
            pltpu.VMEM((CHUNK,), jnp.float32),
            pltpu.VMEM((ZCHUNK,), jnp.float32),
            pltpu.VMEM_SHARED((N,), jnp.float32),
            pltpu.SemaphoreType.DMA,
        ],
    )
    def deg_kernel(dstm_hbm, out_hbm, dstidx, valbuf, zbuf, acc, sem):
        c = lax.axis_index("c")
        s = lax.axis_index("s")
        wid = c * NS + s

        ones16 = jnp.ones((16,), jnp.float32)
        zero16 = jnp.zeros((16,), jnp.float32)

        def fill(i, _):
            valbuf[pl.ds(i * 16, 16)] = ones16
            return 0
        lax.fori_loop(0, CHUNK // 16, fill, 0)

        def zfill(i, _):
            zbuf[pl.ds(i * 16, 16)] = zero16
            return 0
        lax.fori_loop(0, ZCHUNK // 16, zfill, 0)

        pltpu.sync_copy(dstm_hbm.at[wid], dstidx)

        def acc_zero(j, _):
            k = s * 8 + j

            @pl.when(k < RCHUNKS)
            def _():
                pltpu.sync_copy(zbuf, acc.at[pl.ds(k * ZCHUNK, ZCHUNK)])
            return 0
        lax.fori_loop(0, 8, acc_zero, 0)
        plsc.subcore_barrier()

        def fire(g, _):
            pltpu.async_copy(valbuf, acc.at[dstidx.at[g]], sem, add=True)
            return 0
        lax.fori_loop(0, NFULL, fire, 0)

        def drain(g, _):
            pltpu.make_async_copy(valbuf, acc.at[dstidx.at[0]], sem).wait()
            return 0
        lax.fori_loop(0, NFULL, drain, 0)

        plsc.subcore_barrier()

        obase = c * N

        def dump(j, _):
            k = s * 8 + j

            @pl.when(k < RCHUNKS)
            def _():
                pltpu.sync_copy(acc.at[pl.ds(k * ZCHUNK, ZCHUNK)], zbuf)
                pltpu.sync_copy(zbuf, out_hbm.at[pl.ds(obase + k * ZCHUNK, ZCHUNK)])
            return 0
        lax.fori_loop(0, 8, dump, 0)

    return deg_kernel(dstm)


def _sc_scatter(table, srcm, dstmA, dstmB):
    @functools.partial(
        pl.kernel,
        out_type=jax.ShapeDtypeStruct((NC, N, H), jnp.float32),
        mesh=_SC_MESH,
        scratch_types=[
            pltpu.VMEM((EPW,), jnp.int32),
            pltpu.VMEM((PH0, CHUNK), jnp.int32),
            [pltpu.VMEM((CHUNK, H), jnp.float32)] * NBUF,
            pltpu.VMEM_SHARED((N, H), jnp.float32),
            [pltpu.SemaphoreType.DMA] * NBUF,
            [pltpu.SemaphoreType.DMA] * NBUF,
        ],
    )
    def scat_kernel(table_hbm, srcm_hbm, dstmA_hbm, dstmB_hbm,
                    out_hbm, srcidx, dstidx, rows, acc, gsems, ssems):
        c = lax.axis_index("c")
        s = lax.axis_index("s")
        wid = c * NS + s

        zero16 = jnp.zeros((16,), jnp.float32)

        pltpu.sync_copy(srcm_hbm.at[pl.ds(wid * EPW, EPW)], srcidx)
        pltpu.sync_copy(dstmA_hbm.at[wid], dstidx)

        def zrow(r, _):
            def zcol(cc, _):
                rows[0][r, pl.ds(cc * 16, 16)] = zero16
                return 0
            lax.fori_loop(0, H // 16, zcol, 0)
            return 0
        lax.fori_loop(0, ZCHUNK, zrow, 0)

        def acc_zero(j, _):
            k = s * 8 + j

            @pl.when(k < RCHUNKS)
            def _():
                pltpu.sync_copy(rows[0].at[pl.ds(0, ZCHUNK), :],
                                acc.at[pl.ds(k * ZCHUNK, ZCHUNK), :])
            return 0
        lax.fori_loop(0, 8, acc_zero, 0)
        plsc.subcore_barrier()

        def gather(g, b):
            pltpu.async_copy(table_hbm.at[srcidx.at[pl.ds(g * CHUNK, CHUNK)]],
                             rows[b], gsems[b])

        def gather_wait(b):
            pltpu.make_async_copy(
                table_hbm.at[srcidx.at[pl.ds(0, CHUNK)]], rows[b],
                gsems[b]).wait()

        def scatter(g, b):
            pltpu.async_copy(rows[b], acc.at[dstidx.at[g]], ssems[b],
                             add=True)

        def scatter_wait(b):
            pltpu.make_async_copy(rows[b], acc.at[dstidx.at[0]],
                                  ssems[b]).wait()

        for b in range(NBUF):
            gather(b, b)

        def round0(r, _):
            for b in range(NBUF):
                gather_wait(b)
                scatter(NBUF * r + b, b)
            for b in range(NBUF):
                scatter_wait(b)
                gather(NBUF * r + NBUF + b, b)
            return 0
        lax.fori_loop(0, NR0, round0, 0)

        pltpu.sync_copy(dstmB_hbm.at[wid], dstidx.at[pl.ds(0, PH1), :])

        def round1(r, _):
            for b in range(NBUF):
                gather_wait(b)
                scatter(r * NBUF + b, b)

            for b in range(NBUF):
                scatter_wait(b)

                @pl.when(PH0 + r * NBUF + NBUF + b < NFULL)
                def _():
                    gather(PH0 + r * NBUF + NBUF + b, b)
            return 0
        lax.fori_loop(0, NR1, round1, 0)

        for i in range(NLEFT):
            b = i
            gather_wait(b)
            scatter(NR1 * NBUF + i, b)
            scatter_wait(b)
        plsc.subcore_barrier()

        def dump(j, _):
            k = s * 8 + j

            @pl.when(k < RCHUNKS)
            def _():
                pltpu.sync_copy(acc.at[pl.ds(k * ZCHUNK, ZCHUNK), :],
                                rows[0].at[pl.ds(0, ZCHUNK), :])
                pltpu.sync_copy(rows[0].at[pl.ds(0, ZCHUNK), :],
                                out_hbm.at[c, pl.ds(k * ZCHUNK, ZCHUNK), :])
            return 0
        lax.fori_loop(0, 8, dump, 0)

    return scat_kernel(table, srcm, dstmA, dstmB)


_BLK = 1000
_NBLK = N // _BLK


def _tc_prescale(x, W1, degp):
    def body(x_ref, w_ref, dp_ref, hw_ref, dinv_ref):
        deg = 1.0 + dp_ref[0] + dp_ref[1]
        dinv = lax.rsqrt(deg)
        dinv_ref[...] = dinv
        hw_ref[...] = jnp.dot(x_ref[...], w_ref[...],
                              preferred_element_type=jnp.float32) * dinv

    return pl.pallas_call(
        body,
        grid=(_NBLK,),
        in_specs=[
            pl.BlockSpec((_BLK, D), lambda i: (i, 0)),
            pl.BlockSpec((D, H), lambda i: (0, 0)),
            pl.BlockSpec((NC, _BLK, 1), lambda i: (0, i, 0)),
        ],
        out_specs=[
            pl.BlockSpec((_BLK, H), lambda i: (i, 0)),
            pl.BlockSpec((_BLK, 1), lambda i: (i, 0)),
        ],
        out_shape=[
            jax.ShapeDtypeStruct((N, H), jnp.float32),
            jax.ShapeDtypeStruct((N, 1), jnp.float32),
        ],
    )(x, W1, degp)


def _tc_layer_mid(Sp, hwp, dinv, b, W2):
    def body(s_ref, hw_ref, dinv_ref, b_ref, w_ref, out_ref):
        dinv = dinv_ref[...]
        h = s_ref[0] + s_ref[1] + hw_ref[...]
        h = jnp.maximum(dinv * h + b_ref[...], 0.0)
        out_ref[...] = jnp.dot(h, w_ref[...],
                               preferred_element_type=jnp.float32) * dinv

    return pl.pallas_call(
        body,
        grid=(_NBLK,),
        in_specs=[
            pl.BlockSpec((NC, _BLK, H), lambda i: (0, i, 0)),
            pl.BlockSpec((_BLK, H), lambda i: (i, 0)),
            pl.BlockSpec((_BLK, 1), lambda i: (i, 0)),
            pl.BlockSpec((1, H), lambda i: (0, 0)),
            pl.BlockSpec((H, H), lambda i: (0, 0)),
        ],
        out_specs=pl.BlockSpec((_BLK, H), lambda i: (i, 0)),
        out_shape=jax.ShapeDtypeStruct((N, H), jnp.float32),
    )(Sp, hwp, dinv, b, W2)


def _tc_finish_pool(Sp, hwp, dinv, b, batch2d):
    def body(s_ref, hw_ref, dinv_ref, b_ref, bat_ref, out_ref, cnt_ref):
        i = pl.program_id(0)
        dinv = dinv_ref[...]
        h = s_ref[0] + s_ref[1] + hw_ref[...]
        h = jnp.maximum(dinv * h + b_ref[...], 0.0)

        gids = lax.broadcasted_iota(jnp.int32, (_BLK, G), 1)
        onehot = (bat_ref[...] == gids).astype(jnp.float32)
        part = lax.dot_general(onehot, h, (((0,), (0,)), ((), ())),
                               preferred_element_type=jnp.float32)
        pcnt = lax.dot_general(onehot, jnp.ones((_BLK, 1), jnp.float32),
                               (((0,), (0,)), ((), ())),
                               preferred_element_type=jnp.float32)

        @pl.when(i == 0)
        def _():
            out_ref[...] = jnp.zeros_like(out_ref)
            cnt_ref[...] = jnp.zeros_like(cnt_ref)

        out_ref[...] += part
        cnt_ref[...] += pcnt

        @pl.when(i == _NBLK - 1)
        def _():
            out_ref[...] = out_ref[...] / jnp.maximum(cnt_ref[...], 1.0)

    return pl.pallas_call(
        body,
        grid=(_NBLK,),
        in_specs=[
            pl.BlockSpec((NC, _BLK, H), lambda i: (0, i, 0)),
            pl.BlockSpec((_BLK, H), lambda i: (i, 0)),
            pl.BlockSpec((_BLK, 1), lambda i: (i, 0)),
            pl.BlockSpec((1, H), lambda i: (0, 0)),
            pl.BlockSpec((_BLK, 1), lambda i: (i, 0)),
        ],
        out_specs=pl.BlockSpec((G, H), lambda i: (0, 0)),
        out_shape=jax.ShapeDtypeStruct((G, H), jnp.float32),
        scratch_shapes=[pltpu.VMEM((G, 1), jnp.float32)],
    )(Sp, hwp, dinv, b, batch2d)


def kernel(x, edge_index, batch, W1, b1, W2, b2):
    srcf = edge_index[0]
    dstm = edge_index[1].reshape(NW, NFULL, CHUNK)
    dstmA = dstm[:, :PH0, :]
    dstmB = dstm[:, PH0:, :]

    degp = _sc_degree(dstm)
    degp3 = degp.reshape(NC, N, 1)

    hw1p, dinv = _tc_prescale(x, W1, degp3)
    S1 = _sc_scatter(hw1p, srcf, dstmA, dstmB)
    hw2p = _tc_layer_mid(S1, hw1p, dinv, b1.reshape(1, H), W2)
    S2 = _sc_scatter(hw2p, srcf, dstmA, dstmB)
    g = _tc_finish_pool(S2, hw2p, dinv, b2.reshape(1, H),
                        batch.reshape(N, 1))
    return g
